# baseline (speedup 1.0000x reference)
import functools

import jax
import jax.numpy as jnp
from jax import lax
from jax.experimental import pallas as pl
from jax.experimental.pallas import tpu as pltpu
from jax.experimental.pallas import tpu_sc as plsc

F32 = jnp.float32


def _precompute_body(h, w1a, w1b, be1, outa, outb):
    hv = h[...]
    outa[...] = jnp.dot(hv, w1a[...], preferred_element_type=F32) + be1[...]
    outb[...] = jnp.dot(hv, w1b[...], preferred_element_type=F32)


def _edge_body(ga, gb, r2, ea, w1e, w1r, w2, b2, wat, ba, out):
    radial = jnp.transpose(jnp.sqrt(r2[...])[0])
    z = (ga[...] + gb[...] + radial * w1r[...]
         + jnp.dot(ea[...], w1e[...], preferred_element_type=F32))
    m = z * jax.nn.sigmoid(z)
    y = jnp.dot(m, w2[...], preferred_element_type=F32) + b2[...]
    m2 = y * jax.nn.sigmoid(y)
    att_logit = jnp.sum(m2 * wat[...], axis=1, keepdims=True) + ba[...]
    out[...] = m2 * jax.nn.sigmoid(att_logit)


def _node_body(h, s0, s1, wh1a, wh1b, bh1, wh2, bh2, out):
    hv = h[...]
    s = s0[...] + s1[...]
    t = (jnp.dot(hv, wh1a[...], preferred_element_type=F32)
         + jnp.dot(s, wh1b[...], preferred_element_type=F32) + bh1[...])
    t = t * jax.nn.sigmoid(t)
    out[...] = hv + jnp.dot(t, wh2[...], preferred_element_type=F32) + bh2[...]


def _pick_chunk(epw, step, cap=128):
    for c in range(cap - cap % step, 0, -step):
        if epw % c == 0:
            return c
    raise ValueError(epw)


def _make_gather(n, e, d):
    info = plsc.get_sparse_core_info()
    nc, ns, nl = info.num_cores, info.num_subcores, info.num_lanes
    nw = nc * ns
    epw = e // nw
    chunk = _pick_chunk(epw, nl, 80)
    nchunk = epw // chunk
    nquads = nchunk // 4
    groups = chunk // nl
    nslot = 4
    mesh = plsc.VectorSubcoreMesh(core_axis_name="c", subcore_axis_name="s")

    @functools.partial(
        pl.kernel, mesh=mesh,
        out_type=[jax.ShapeDtypeStruct((e, d), F32),
                  jax.ShapeDtypeStruct((e, d), F32),
                  jax.ShapeDtypeStruct((e,), F32)],
        scratch_types=[pltpu.VMEM((nslot, chunk), jnp.int32),
                       pltpu.VMEM((nslot, chunk), jnp.int32),
                       pltpu.VMEM((chunk, d), F32),
                       pltpu.VMEM((chunk, d), F32),
                       pltpu.VMEM((chunk, d), F32),
                       pltpu.VMEM((chunk, d), F32),
                       pltpu.VMEM((chunk, d), F32),
                       pltpu.VMEM((chunk, d), F32),
                       pltpu.VMEM((chunk, d), F32),
                       pltpu.VMEM((chunk, d), F32),
                       pltpu.VMEM((nslot, chunk), F32),
                       pltpu.VMEM((n,), F32),
                       pltpu.VMEM((n,), F32),
                       pltpu.VMEM((n,), F32)]
                      + [pltpu.SemaphoreType.DMA] * (5 * nslot),
        compiler_params=pltpu.CompilerParams(needs_layout_passes=False),
    )
    def gather_k(ta, tb, row, col, x0, x1, x2, outa, outb, outr2,
                 idxrm, idxcm, bufa0, bufa1, bufa2, bufa3,
                 bufb0, bufb1, bufb2, bufb3, r2m, xa, xb, xc, *sems):
        idxr = [idxrm.at[i] for i in range(nslot)]
        idxc = [idxcm.at[i] for i in range(nslot)]
        bufa = [bufa0, bufa1, bufa2, bufa3]
        bufb = [bufb0, bufb1, bufb2, bufb3]
        r2b = [r2m.at[i] for i in range(nslot)]
        sga = list(sems[0:nslot])
        sgb = list(sems[nslot:2 * nslot])
        swa = list(sems[2 * nslot:3 * nslot])
        swb = list(sems[3 * nslot:4 * nslot])
        swr = list(sems[4 * nslot:5 * nslot])

        wid = lax.axis_index("s") * nc + lax.axis_index("c")
        base = wid * epw
        pltpu.sync_copy(x0, xa)
        pltpu.sync_copy(x1, xb)
        pltpu.sync_copy(x2, xc)

        def issue(k, s):
            cb = pl.multiple_of(base + k * chunk, 8)
            pltpu.sync_copy(row.at[pl.ds(cb, chunk)], idxr[s])
            pltpu.sync_copy(col.at[pl.ds(cb, chunk)], idxc[s])
            pltpu.async_copy(ta.at[idxr[s]], bufa[s], sga[s])
            pltpu.async_copy(tb.at[idxc[s]], bufb[s], sgb[s])

        def wait_gathers(s):
            pltpu.make_async_copy(ta.at[idxr[s]], bufa[s], sga[s]).wait()
            pltpu.make_async_copy(tb.at[idxc[s]], bufb[s], sgb[s]).wait()

        def r2comp(s):
            for g in range(groups):
                ir = idxr[s][pl.ds(g * nl, nl)]
                ic = idxc[s][pl.ds(g * nl, nl)]
                dx = plsc.load_gather(xa, [ir]) - plsc.load_gather(xa, [ic])
                dy = plsc.load_gather(xb, [ir]) - plsc.load_gather(xb, [ic])
                dz = plsc.load_gather(xc, [ir]) - plsc.load_gather(xc, [ic])
                r2b[s][pl.ds(g * nl, nl)] = dx * dx + dy * dy + dz * dz

        def flush(k, s):
            cb = pl.multiple_of(base + k * chunk, 8)
            pltpu.async_copy(bufa[s], outa.at[pl.ds(cb, chunk)], swa[s])
            pltpu.async_copy(bufb[s], outb.at[pl.ds(cb, chunk)], swb[s])
            pltpu.async_copy(r2b[s], outr2.at[pl.ds(cb, chunk)], swr[s])

        def wait_flush(s):
            z2 = pl.ds(0, chunk)
            pltpu.make_async_copy(bufa[s], outa.at[z2], swa[s]).wait()
            pltpu.make_async_copy(bufb[s], outb.at[z2], swb[s]).wait()
            pltpu.make_async_copy(r2b[s], outr2.at[z2], swr[s]).wait()

        issue(0, 0)

        def body(q, carry):
            for i in range(4):
                k = 4 * q + i
                sn = (i + 1) % 4
                if i == 3:
                    wait_flush(0)
                else:
                    pl.when(q > 0)(lambda sn=sn: wait_flush(sn))
                pl.when(k + 1 < nchunk)(lambda k=k, sn=sn: issue(k + 1, sn))
                wait_gathers(i)
                r2comp(i)
                flush(k, i)
            return carry

        lax.fori_loop(0, nquads, body, 0)
        rest = nchunk - 4 * nquads
        for i in range(rest):
            k = 4 * nquads + i
            sn = (i + 1) % 4
            if k + 1 < nchunk:
                wait_flush(sn)
                issue(k + 1, sn)
            wait_gathers(i)
            r2comp(i)
            flush(k, i)
        for k in range(max(0, nchunk - 4), nchunk):
            wait_flush(k % 4)

    return gather_k


def _make_scatter(n, e, d):
    info = plsc.get_sparse_core_info()
    nc, ns = info.num_cores, info.num_subcores
    nw = nc * ns
    epw = e // nw
    chunk = _pick_chunk(epw, 8)
    nchunk = epw // chunk
    npairs = (nchunk - 1) // 2
    rps = -(-n // (8 * ns)) * 8
    npad = rps * ns
    mesh = plsc.VectorSubcoreMesh(core_axis_name="c", subcore_axis_name="s")

    @functools.partial(
        pl.kernel, mesh=mesh,
        out_type=jax.ShapeDtypeStruct((nc * npad, d), F32),
        scratch_types=[pltpu.VMEM((chunk,), jnp.int32),
                       pltpu.VMEM((chunk,), jnp.int32),
                       pltpu.VMEM((chunk, d), F32),
                       pltpu.VMEM((chunk, d), F32),
                       pltpu.VMEM_SHARED((npad, d), F32)]
                      + [pltpu.SemaphoreType.DMA] * 4,
    )
    def scatter_k(msg, row, init, out, idx0, idx1, mb0, mb1, acc,
                  si0, si1, sm0, sm1):
        idxv, mbuf = [idx0, idx1], [mb0, mb1]
        si, sm = [si0, si1], [sm0, sm1]
        c = lax.axis_index("c")
        s = lax.axis_index("s")
        wid = s * nc + c
        pltpu.sync_copy(
            init.at[pl.ds(pl.multiple_of(c * npad + s * rps, 8), rps)],
            acc.at[pl.ds(pl.multiple_of(s * rps, 8), rps)])
        plsc.subcore_barrier()
        base = wid * epw

        def load(k, sl):
            cb = pl.multiple_of(base + k * chunk, 8)
            pltpu.async_copy(row.at[pl.ds(cb, chunk)], idxv[sl], si[sl])
            pltpu.async_copy(msg.at[pl.ds(cb, chunk)], mbuf[sl], sm[sl])

        def wait_load(sl):
            z1 = pl.ds(0, chunk)
            pltpu.make_async_copy(row.at[z1], idxv[sl], si[sl]).wait()
            pltpu.make_async_copy(msg.at[z1], mbuf[sl], sm[sl]).wait()

        def add(sl):
            pltpu.sync_copy(mbuf[sl], acc.at[idxv[sl]], add=True)

        load(0, 0)

        def body(j2, carry):
            p0 = 2 * j2
            load(p0 + 1, 1)
            wait_load(0)
            add(0)
            pl.when(j2 < npairs - 1)(lambda: load(p0 + 2, 0))
            wait_load(1)
            add(1)
            return carry

        lax.fori_loop(0, npairs, body, 0)
        rest = nchunk - 2 * npairs
        if rest == 2:
            load(nchunk - 2, 0)
            load(nchunk - 1, 1)
            wait_load(0)
            add(0)
            wait_load(1)
            add(1)
        else:
            load(nchunk - 1, 0)
            wait_load(0)
            add(0)
        plsc.subcore_barrier()
        pltpu.sync_copy(acc.at[pl.ds(pl.multiple_of(s * rps, 8), rps)],
                        out.at[pl.ds(pl.multiple_of(c * npad + s * rps, 8),
                                     rps)])

    return scatter_k, npad


def kernel(x, h, edge_index, edge_attr, We1, be1, We2, be2, Wa, ba,
           Wh1, bh1, Wh2, bh2):
    n, d = h.shape
    e = edge_attr.shape[0]
    de = edge_attr.shape[1]

    row = edge_index[0].astype(jnp.int32)
    col = edge_index[1].astype(jnp.int32)
    xf = x.astype(F32)

    w1a = We1[:d]
    w1b = We1[d:2 * d]
    w1r = We1[2 * d:2 * d + 1]
    w1e = We1[2 * d + 1:]

    nb = 2000
    grid_n = n // nb
    full = lambda shape: pl.BlockSpec(shape, lambda i: tuple(0 for _ in shape))
    rowblk = lambda r, c_: pl.BlockSpec((r, c_), lambda i: (i, 0))

    ta, tb = pl.pallas_call(
        _precompute_body,
        grid=(grid_n,),
        in_specs=[rowblk(nb, d), full((d, 128)), full((d, 128)),
                  full((1, 128))],
        out_specs=[rowblk(nb, 128), rowblk(nb, 128)],
        out_shape=[jax.ShapeDtypeStruct((n, 128), F32),
                   jax.ShapeDtypeStruct((n, 128), F32)],
    )(h, w1a, w1b, be1.reshape(1, 128))

    eb = 2560
    units = e // eb
    nseg = 3
    seg_units = [units // nseg + (1 if i < units % nseg else 0)
                 for i in range(nseg)]
    partials = None
    npad = None
    off = 0
    for su in seg_units:
        sz = su * eb
        rs = row[off:off + sz]
        cs = col[off:off + sz]
        ga, gb, r2 = _make_gather(n, sz, 128)(
            ta, tb, rs, cs, xf[:, 0], xf[:, 1], xf[:, 2])
        msg = pl.pallas_call(
            _edge_body,
            grid=(su,),
            in_specs=[rowblk(eb, 128), rowblk(eb, 128),
                      pl.BlockSpec((1, 1, eb), lambda i: (i, 0, 0)),
                      rowblk(eb, de), full((de, 128)), full((1, 128)),
                      full((128, 128)), full((1, 128)), full((1, 128)),
                      full((1, 1))],
            out_specs=rowblk(eb, 128),
            out_shape=jax.ShapeDtypeStruct((sz, 128), F32),
        )(ga, gb, r2.reshape(su, 1, eb), edge_attr[off:off + sz], w1e, w1r,
          We2, be2.reshape(1, 128), Wa.reshape(1, 128), ba.reshape(1, 1))
        scatter_k, npad = _make_scatter(n, sz, 128)
        init = (jnp.zeros((2 * npad, 128), F32) if partials is None
                else partials)
        partials = scatter_k(msg, rs, init)
        off += sz
    s0 = partials[:n]
    s1 = partials[npad:npad + n]

    out = pl.pallas_call(
        _node_body,
        grid=(grid_n,),
        in_specs=[rowblk(nb, d), rowblk(nb, 128), rowblk(nb, 128),
                  full((128, 128)), full((128, 128)), full((1, 128)),
                  full((128, 128)), full((1, 128))],
        out_specs=rowblk(nb, d),
        out_shape=jax.ShapeDtypeStruct((n, d), F32),
    )(h, s0, s1, Wh1[:d], Wh1[d:], bh1.reshape(1, 128), Wh2,
      bh2.reshape(1, 128))

    return out


# speedup vs baseline: 1.0028x; 1.0028x over previous
import functools

import jax
import jax.numpy as jnp
from jax import lax
from jax.experimental import pallas as pl
from jax.experimental.pallas import tpu as pltpu
from jax.experimental.pallas import tpu_sc as plsc

F32 = jnp.float32


def _precompute_body(h, w1a, w1b, be1, outa, outb):
    hv = h[...]
    outa[...] = jnp.dot(hv, w1a[...], preferred_element_type=F32) + be1[...]
    outb[...] = jnp.dot(hv, w1b[...], preferred_element_type=F32)


def _edge_body(ga, gb, r2, ea, w1e, w1r, w2, b2, wat, ba, out):
    radial = jnp.transpose(jnp.sqrt(r2[...])[0])
    z = (ga[...] + gb[...] + radial * w1r[...]
         + jnp.dot(ea[...], w1e[...], preferred_element_type=F32))
    m = z * jax.nn.sigmoid(z)
    y = jnp.dot(m, w2[...], preferred_element_type=F32) + b2[...]
    m2 = y * jax.nn.sigmoid(y)
    att_logit = jnp.sum(m2 * wat[...], axis=1, keepdims=True) + ba[...]
    out[...] = m2 * jax.nn.sigmoid(att_logit)


def _node_body(h, s0, s1, wh1a, wh1b, bh1, wh2, bh2, out):
    hv = h[...]
    s = s0[...] + s1[...]
    t = (jnp.dot(hv, wh1a[...], preferred_element_type=F32)
         + jnp.dot(s, wh1b[...], preferred_element_type=F32) + bh1[...])
    t = t * jax.nn.sigmoid(t)
    out[...] = hv + jnp.dot(t, wh2[...], preferred_element_type=F32) + bh2[...]


def _pick_chunk(epw, step, cap=128):
    for c in range(cap - cap % step, 0, -step):
        if epw % c == 0:
            return c
    raise ValueError(epw)


def _make_gather(n, e, d):
    info = plsc.get_sparse_core_info()
    nc, ns, nl = info.num_cores, info.num_subcores, info.num_lanes
    nw = nc * ns
    epw = e // nw
    chunk = _pick_chunk(epw, nl, 80)
    nchunk = epw // chunk
    nquads = nchunk // 4
    groups = chunk // nl
    nslot = 4
    mesh = plsc.VectorSubcoreMesh(core_axis_name="c", subcore_axis_name="s")

    @functools.partial(
        pl.kernel, mesh=mesh,
        out_type=[jax.ShapeDtypeStruct((e, d), F32),
                  jax.ShapeDtypeStruct((e, d), F32),
                  jax.ShapeDtypeStruct((e,), F32)],
        scratch_types=[pltpu.VMEM((nslot, chunk), jnp.int32),
                       pltpu.VMEM((nslot, chunk), jnp.int32),
                       pltpu.VMEM((chunk, d), F32),
                       pltpu.VMEM((chunk, d), F32),
                       pltpu.VMEM((chunk, d), F32),
                       pltpu.VMEM((chunk, d), F32),
                       pltpu.VMEM((chunk, d), F32),
                       pltpu.VMEM((chunk, d), F32),
                       pltpu.VMEM((chunk, d), F32),
                       pltpu.VMEM((chunk, d), F32),
                       pltpu.VMEM((nslot, chunk), F32),
                       pltpu.VMEM((n,), F32),
                       pltpu.VMEM((n,), F32),
                       pltpu.VMEM((n,), F32)]
                      + [pltpu.SemaphoreType.DMA] * (5 * nslot),
        compiler_params=pltpu.CompilerParams(needs_layout_passes=False),
    )
    def gather_k(ta, tb, row, col, x0, x1, x2, outa, outb, outr2,
                 idxrm, idxcm, bufa0, bufa1, bufa2, bufa3,
                 bufb0, bufb1, bufb2, bufb3, r2m, xa, xb, xc, *sems):
        idxr = [idxrm.at[i] for i in range(nslot)]
        idxc = [idxcm.at[i] for i in range(nslot)]
        bufa = [bufa0, bufa1, bufa2, bufa3]
        bufb = [bufb0, bufb1, bufb2, bufb3]
        r2b = [r2m.at[i] for i in range(nslot)]
        sga = list(sems[0:nslot])
        sgb = list(sems[nslot:2 * nslot])
        swa = list(sems[2 * nslot:3 * nslot])
        swb = list(sems[3 * nslot:4 * nslot])
        swr = list(sems[4 * nslot:5 * nslot])

        wid = lax.axis_index("s") * nc + lax.axis_index("c")
        base = wid * epw
        pltpu.sync_copy(x0, xa)
        pltpu.sync_copy(x1, xb)
        pltpu.sync_copy(x2, xc)

        def issue(k, s):
            cb = pl.multiple_of(base + k * chunk, 8)
            pltpu.sync_copy(row.at[pl.ds(cb, chunk)], idxr[s])
            pltpu.sync_copy(col.at[pl.ds(cb, chunk)], idxc[s])
            pltpu.async_copy(ta.at[idxr[s]], bufa[s], sga[s])
            pltpu.async_copy(tb.at[idxc[s]], bufb[s], sgb[s])

        def wait_gathers(s):
            pltpu.make_async_copy(ta.at[idxr[s]], bufa[s], sga[s]).wait()
            pltpu.make_async_copy(tb.at[idxc[s]], bufb[s], sgb[s]).wait()

        def r2comp(s):
            for g in range(groups):
                ir = idxr[s][pl.ds(g * nl, nl)]
                ic = idxc[s][pl.ds(g * nl, nl)]
                dx = plsc.load_gather(xa, [ir]) - plsc.load_gather(xa, [ic])
                dy = plsc.load_gather(xb, [ir]) - plsc.load_gather(xb, [ic])
                dz = plsc.load_gather(xc, [ir]) - plsc.load_gather(xc, [ic])
                r2b[s][pl.ds(g * nl, nl)] = dx * dx + dy * dy + dz * dz

        def flush(k, s):
            cb = pl.multiple_of(base + k * chunk, 8)
            pltpu.async_copy(bufa[s], outa.at[pl.ds(cb, chunk)], swa[s])
            pltpu.async_copy(bufb[s], outb.at[pl.ds(cb, chunk)], swb[s])
            pltpu.async_copy(r2b[s], outr2.at[pl.ds(cb, chunk)], swr[s])

        def wait_flush(s):
            z2 = pl.ds(0, chunk)
            pltpu.make_async_copy(bufa[s], outa.at[z2], swa[s]).wait()
            pltpu.make_async_copy(bufb[s], outb.at[z2], swb[s]).wait()
            pltpu.make_async_copy(r2b[s], outr2.at[z2], swr[s]).wait()

        issue(0, 0)

        def body(q, carry):
            for i in range(4):
                k = 4 * q + i
                sn = (i + 1) % 4
                if i == 3:
                    wait_flush(0)
                else:
                    pl.when(q > 0)(lambda sn=sn: wait_flush(sn))
                pl.when(k + 1 < nchunk)(lambda k=k, sn=sn: issue(k + 1, sn))
                wait_gathers(i)
                r2comp(i)
                flush(k, i)
            return carry

        lax.fori_loop(0, nquads, body, 0)
        rest = nchunk - 4 * nquads
        for i in range(rest):
            k = 4 * nquads + i
            sn = (i + 1) % 4
            if k + 1 < nchunk:
                wait_flush(sn)
                issue(k + 1, sn)
            wait_gathers(i)
            r2comp(i)
            flush(k, i)
        for k in range(max(0, nchunk - 4), nchunk):
            wait_flush(k % 4)

    return gather_k


def _make_scatter(n, e, d):
    info = plsc.get_sparse_core_info()
    nc, ns = info.num_cores, info.num_subcores
    nw = nc * ns
    epw = e // nw
    chunk = _pick_chunk(epw, 8)
    nchunk = epw // chunk
    npairs = (nchunk - 1) // 2
    rps = -(-n // (8 * ns)) * 8
    npad = rps * ns
    mesh = plsc.VectorSubcoreMesh(core_axis_name="c", subcore_axis_name="s")

    @functools.partial(
        pl.kernel, mesh=mesh,
        out_type=jax.ShapeDtypeStruct((nc * npad, d), F32),
        scratch_types=[pltpu.VMEM((chunk,), jnp.int32),
                       pltpu.VMEM((chunk,), jnp.int32),
                       pltpu.VMEM((chunk, d), F32),
                       pltpu.VMEM((chunk, d), F32),
                       pltpu.VMEM_SHARED((npad, d), F32)]
                      + [pltpu.SemaphoreType.DMA] * 4,
    )
    def scatter_k(msg, row, init, out, idx0, idx1, mb0, mb1, acc,
                  si0, si1, sm0, sm1):
        idxv, mbuf = [idx0, idx1], [mb0, mb1]
        si, sm = [si0, si1], [sm0, sm1]
        c = lax.axis_index("c")
        s = lax.axis_index("s")
        wid = s * nc + c
        pltpu.sync_copy(
            init.at[pl.ds(pl.multiple_of(c * npad + s * rps, 8), rps)],
            acc.at[pl.ds(pl.multiple_of(s * rps, 8), rps)])
        plsc.subcore_barrier()
        base = wid * epw

        def load(k, sl):
            cb = pl.multiple_of(base + k * chunk, 8)
            pltpu.async_copy(row.at[pl.ds(cb, chunk)], idxv[sl], si[sl])
            pltpu.async_copy(msg.at[pl.ds(cb, chunk)], mbuf[sl], sm[sl])

        def wait_load(sl):
            z1 = pl.ds(0, chunk)
            pltpu.make_async_copy(row.at[z1], idxv[sl], si[sl]).wait()
            pltpu.make_async_copy(msg.at[z1], mbuf[sl], sm[sl]).wait()

        def add(sl):
            pltpu.sync_copy(mbuf[sl], acc.at[idxv[sl]], add=True)

        load(0, 0)

        def body(j2, carry):
            p0 = 2 * j2
            load(p0 + 1, 1)
            wait_load(0)
            add(0)
            pl.when(j2 < npairs - 1)(lambda: load(p0 + 2, 0))
            wait_load(1)
            add(1)
            return carry

        lax.fori_loop(0, npairs, body, 0)
        rest = nchunk - 2 * npairs
        if rest == 2:
            load(nchunk - 2, 0)
            load(nchunk - 1, 1)
            wait_load(0)
            add(0)
            wait_load(1)
            add(1)
        else:
            load(nchunk - 1, 0)
            wait_load(0)
            add(0)
        plsc.subcore_barrier()
        pltpu.sync_copy(acc.at[pl.ds(pl.multiple_of(s * rps, 8), rps)],
                        out.at[pl.ds(pl.multiple_of(c * npad + s * rps, 8),
                                     rps)])

    return scatter_k, npad


def kernel(x, h, edge_index, edge_attr, We1, be1, We2, be2, Wa, ba,
           Wh1, bh1, Wh2, bh2):
    n, d = h.shape
    e = edge_attr.shape[0]
    de = edge_attr.shape[1]

    row = edge_index[0].astype(jnp.int32)
    col = edge_index[1].astype(jnp.int32)
    xf = x.astype(F32)

    w1a = We1[:d]
    w1b = We1[d:2 * d]
    w1r = We1[2 * d:2 * d + 1]
    w1e = We1[2 * d + 1:]

    nb = 2000
    grid_n = n // nb
    full = lambda shape: pl.BlockSpec(shape, lambda i: tuple(0 for _ in shape))
    rowblk = lambda r, c_: pl.BlockSpec((r, c_), lambda i: (i, 0))

    ta, tb = pl.pallas_call(
        _precompute_body,
        grid=(grid_n,),
        in_specs=[rowblk(nb, d), full((d, 128)), full((d, 128)),
                  full((1, 128))],
        out_specs=[rowblk(nb, 128), rowblk(nb, 128)],
        out_shape=[jax.ShapeDtypeStruct((n, 128), F32),
                   jax.ShapeDtypeStruct((n, 128), F32)],
    )(h, w1a, w1b, be1.reshape(1, 128))

    eb = 2560
    units = e // eb
    u0 = units // 5
    rem = units - u0
    seg_units = [u0, rem - rem // 2, rem // 2]
    partials = None
    npad = None
    off = 0
    for su in seg_units:
        sz = su * eb
        rs = row[off:off + sz]
        cs = col[off:off + sz]
        ga, gb, r2 = _make_gather(n, sz, 128)(
            ta, tb, rs, cs, xf[:, 0], xf[:, 1], xf[:, 2])
        msg = pl.pallas_call(
            _edge_body,
            grid=(su,),
            in_specs=[rowblk(eb, 128), rowblk(eb, 128),
                      pl.BlockSpec((1, 1, eb), lambda i: (i, 0, 0)),
                      rowblk(eb, de), full((de, 128)), full((1, 128)),
                      full((128, 128)), full((1, 128)), full((1, 128)),
                      full((1, 1))],
            out_specs=rowblk(eb, 128),
            out_shape=jax.ShapeDtypeStruct((sz, 128), F32),
        )(ga, gb, r2.reshape(su, 1, eb), edge_attr[off:off + sz], w1e, w1r,
          We2, be2.reshape(1, 128), Wa.reshape(1, 128), ba.reshape(1, 1))
        scatter_k, npad = _make_scatter(n, sz, 128)
        init = (jnp.zeros((2 * npad, 128), F32) if partials is None
                else partials)
        partials = scatter_k(msg, rs, init)
        off += sz
    s0 = partials[:n]
    s1 = partials[npad:npad + n]

    out = pl.pallas_call(
        _node_body,
        grid=(grid_n,),
        in_specs=[rowblk(nb, d), rowblk(nb, 128), rowblk(nb, 128),
                  full((128, 128)), full((128, 128)), full((1, 128)),
                  full((128, 128)), full((1, 128))],
        out_specs=rowblk(nb, d),
        out_shape=jax.ShapeDtypeStruct((n, d), F32),
    )(h, s0, s1, Wh1[:d], Wh1[d:], bh1.reshape(1, 128), Wh2,
      bh2.reshape(1, 128))

    return out
